# Initial kernel scaffold; baseline (speedup 1.0000x reference)
import jax
import jax.numpy as jnp
from jax.experimental import pallas as pl


def kernel(x, edge_index, W1, b1, W2, b2, W3, b3, Wl, bl):
    raise NotImplementedError("write your pallas kernel here")


# speedup vs baseline: 9.4931x; 9.4931x over previous
import functools
import jax
import jax.numpy as jnp
from jax import lax
from jax.experimental import pallas as pl
from jax.experimental.pallas import tpu as pltpu
from jax.experimental.pallas import tpu_sc as plsc

N = 10000
E = 320000
D = 128
H = 128
C = 40

NC = 2
NS = 16
NW = NC * NS
CH = 128
NPAD = 10240
ZROWS = NPAD // NS
OROWS = N // NS
EPT = 10112
EPAD = EPT * NW

def _deg_body(dst_hbm, out_hbm, dst_v, ones_v, zero_v, acc_sh):
    c = lax.axis_index("c")
    s = lax.axis_index("s")
    wid = s * NC + c
    base = wid * EPT

    def _fill(i, _):
        ones_v[pl.ds(pl.multiple_of(i * 16, 16), 16)] = jnp.ones((16,), jnp.float32)
        return 0

    lax.fori_loop(0, CH // 16, _fill, 0)

    def _zfill(i, _):
        zero_v[pl.ds(pl.multiple_of(i * 16, 16), 16)] = jnp.zeros((16,), jnp.float32)
        return 0

    lax.fori_loop(0, ZROWS // 16, _zfill, 0)

    pltpu.sync_copy(zero_v, acc_sh.at[pl.ds(s * ZROWS, ZROWS)])
    plsc.subcore_barrier()

    def _chunk(j, _):
        off = pl.multiple_of(base + j * CH, CH)
        pltpu.sync_copy(dst_hbm.at[pl.ds(off, CH)], dst_v)
        pltpu.sync_copy(ones_v, acc_sh.at[dst_v], add=True)
        return 0

    lax.fori_loop(0, EPT // CH, _chunk, 0)
    plsc.subcore_barrier()
    pltpu.sync_copy(acc_sh.at[pl.ds(s * ZROWS, ZROWS)],
                    out_hbm.at[c, pl.ds(s * ZROWS, ZROWS)])


def _agg_body(g_hbm, src_hbm, dst_hbm, out_hbm,
              src_v, dst_v, rows_v, zero_v, sem, acc_sh):
    c = lax.axis_index("c")
    s = lax.axis_index("s")
    wid = s * NC + c
    base = wid * EPT

    def _zfill(i, _):
        for j in range(H // 16):
            zero_v[i, pl.ds(j * 16, 16)] = jnp.zeros((16,), jnp.float32)
        return 0

    lax.fori_loop(0, CH, _zfill, 0)

    for k in range(ZROWS // CH):
        pltpu.sync_copy(zero_v, acc_sh.at[pl.ds(s * ZROWS + k * CH, CH), :])
    plsc.subcore_barrier()

    def _chunk(j, _):
        off = pl.multiple_of(base + j * CH, CH)
        pltpu.sync_copy(src_hbm.at[pl.ds(off, CH)], src_v)
        pltpu.sync_copy(dst_hbm.at[pl.ds(off, CH)], dst_v)
        pltpu.async_copy(g_hbm.at[src_v], rows_v, sem).wait()
        pltpu.sync_copy(rows_v, acc_sh.at[dst_v], add=True)
        return 0

    lax.fori_loop(0, EPT // CH, _chunk, 0)
    plsc.subcore_barrier()
    pltpu.sync_copy(acc_sh.at[pl.ds(s * ZROWS, ZROWS), :],
                    out_hbm.at[c, pl.ds(s * ZROWS, ZROWS), :])


@functools.cache
def _sc_kernels():
    mesh = plsc.VectorSubcoreMesh(
        core_axis_name="c", subcore_axis_name="s",
        num_cores=NC, num_subcores=NS)
    deg_k = pl.kernel(
        _deg_body,
        out_type=jax.ShapeDtypeStruct((NC, NPAD), jnp.float32),
        mesh=mesh,
        scratch_types=[
            pltpu.VMEM((CH,), jnp.int32),
            pltpu.VMEM((CH,), jnp.float32),
            pltpu.VMEM((ZROWS,), jnp.float32),
            pltpu.VMEM_SHARED((NPAD,), jnp.float32),
        ],
    )
    agg_k = pl.kernel(
        _agg_body,
        out_type=jax.ShapeDtypeStruct((NC, NPAD, H), jnp.float32),
        mesh=mesh,
        scratch_types=[
            pltpu.VMEM((CH,), jnp.int32),
            pltpu.VMEM((CH,), jnp.int32),
            pltpu.VMEM((CH, H), jnp.float32),
            pltpu.VMEM((CH, H), jnp.float32),
            pltpu.SemaphoreType.DMA,
            pltpu.VMEM_SHARED((NPAD, H), jnp.float32),
        ],
    )
    return deg_k, agg_k


_R = 2000
_G = N // _R


def _mm1_body(x_ref, w_ref, dinv_ref, o_ref):
    g = jnp.dot(x_ref[...], w_ref[...], preferred_element_type=jnp.float32)
    o_ref[...] = g * dinv_ref[...]


_mm1 = pl.pallas_call(
    _mm1_body,
    grid=(_G,),
    in_specs=[
        pl.BlockSpec((_R, D), lambda i: (i, 0)),
        pl.BlockSpec((D, H), lambda i: (0, 0)),
        pl.BlockSpec((_R, 1), lambda i: (i, 0)),
    ],
    out_specs=pl.BlockSpec((_R, H), lambda i: (i, 0)),
    out_shape=jax.ShapeDtypeStruct((N, H), jnp.float32),
)


def _mid_body(acc_ref, gs_ref, b_ref, w_ref, dinv_ref, o_ref):
    a = acc_ref[0] + acc_ref[1] + gs_ref[...]
    h = jnp.maximum(a * dinv_ref[...] + b_ref[0:1, :], 0.0)
    g = jnp.dot(h, w_ref[...], preferred_element_type=jnp.float32)
    o_ref[...] = g * dinv_ref[...]


_mid = pl.pallas_call(
    _mid_body,
    grid=(_G,),
    in_specs=[
        pl.BlockSpec((NC, _R, H), lambda i: (0, i, 0)),
        pl.BlockSpec((_R, H), lambda i: (i, 0)),
        pl.BlockSpec((8, H), lambda i: (0, 0)),
        pl.BlockSpec((H, H), lambda i: (0, 0)),
        pl.BlockSpec((_R, 1), lambda i: (i, 0)),
    ],
    out_specs=pl.BlockSpec((_R, H), lambda i: (i, 0)),
    out_shape=jax.ShapeDtypeStruct((N, H), jnp.float32),
)


def _fin_body(acc_ref, gs_ref, b_ref, w_ref, bl_ref, dinv_ref, o_ref):
    a = acc_ref[0] + acc_ref[1] + gs_ref[...]
    h = jnp.maximum(a * dinv_ref[...] + b_ref[0:1, :], 0.0)
    o_ref[...] = jnp.dot(h, w_ref[...],
                         preferred_element_type=jnp.float32) + bl_ref[0:1, :]


_fin = pl.pallas_call(
    _fin_body,
    grid=(_G,),
    in_specs=[
        pl.BlockSpec((NC, _R, H), lambda i: (0, i, 0)),
        pl.BlockSpec((_R, H), lambda i: (i, 0)),
        pl.BlockSpec((8, H), lambda i: (0, 0)),
        pl.BlockSpec((H, C), lambda i: (0, 0)),
        pl.BlockSpec((8, C), lambda i: (0, 0)),
        pl.BlockSpec((_R, 1), lambda i: (i, 0)),
    ],
    out_specs=pl.BlockSpec((_R, C), lambda i: (i, 0)),
    out_shape=jax.ShapeDtypeStruct((N, C), jnp.float32),
)


def kernel(x, edge_index, W1, b1, W2, b2, W3, b3, Wl, bl):
    src = jnp.concatenate(
        [edge_index[0].astype(jnp.int32),
         jnp.zeros((EPAD - E,), jnp.int32)])
    dst = jnp.concatenate(
        [edge_index[1].astype(jnp.int32),
         jnp.full((EPAD - E,), N, jnp.int32)])

    deg_k, agg_k = _sc_kernels()
    degp = deg_k(dst)
    deg = degp[0, :N] + degp[1, :N] + 1.0
    dinv = lax.rsqrt(jnp.maximum(deg, 1.0)).reshape(N, 1)

    b1t = jnp.tile(b1.reshape(1, H), (8, 1))
    b2t = jnp.tile(b2.reshape(1, H), (8, 1))
    b3t = jnp.tile(b3.reshape(1, H), (8, 1))
    blt = jnp.tile(bl.reshape(1, C), (8, 1))

    gs1 = _mm1(x, W1, dinv)
    acc1 = agg_k(gs1, src, dst)
    gs2 = _mid(acc1, gs1, b1t, W2, dinv)
    acc2 = agg_k(gs2, src, dst)
    gs3 = _mid(acc2, gs2, b2t, W3, dinv)
    acc3 = agg_k(gs3, src, dst)
    return _fin(acc3, gs3, b3t, Wl, blt, dinv)
